# baseline (speedup 1.0000x reference)
import functools

import jax
import jax.numpy as jnp
from jax import lax
from jax.experimental import pallas as pl
from jax.experimental.pallas import tpu as pltpu
from jax.experimental.pallas import tpu_sc as plsc

N = 10000
NP = 10240
E = 80000
NC = 2
NS = 16
NW = NC * NS
ENDP = 2 * E
PER_TILE = 5120
PADDED = NW * PER_TILE
CHUNK = 128
NCHUNK = PER_TILE // CHUNK
GROUPS = CHUNK // 16


def _make_scatter(value: float):
  mesh = plsc.VectorSubcoreMesh(
      core_axis_name="c", subcore_axis_name="s",
      num_cores=NC, num_subcores=NS)

  @functools.partial(
      pl.kernel,
      out_type=(),
      mesh=mesh,
      scratch_types=[
          pltpu.VMEM((PER_TILE,), jnp.int32),
          pltpu.VMEM((PER_TILE,), jnp.int32),
          pltpu.VMEM((NCHUNK, CHUNK), jnp.int32),
          pltpu.VMEM((CHUNK,), jnp.float32),
          pltpu.SemaphoreType.DMA,
      ],
  )
  def scatter(rows_hbm, cols_hbm, m_hbm, rows_v, cols_v, idx_v, val_v, sem):
    wid = lax.axis_index("s") * NC + lax.axis_index("c")
    base = wid * PER_TILE
    pltpu.sync_copy(rows_hbm.at[pl.ds(base, PER_TILE)], rows_v)
    pltpu.sync_copy(cols_hbm.at[pl.ds(base, PER_TILE)], cols_v)

    vval = jnp.full((16,), value, dtype=jnp.float32)
    for g in range(GROUPS):
      val_v[pl.ds(g * 16, 16)] = vval

    @pl.loop(0, NCHUNK)
    def _compute(j):
      for g in range(GROUPS):
        off = j * CHUNK + g * 16
        r = rows_v[pl.ds(off, 16)]
        c = cols_v[pl.ds(off, 16)]
        idx_v[j, pl.ds(g * 16, 16)] = r * NP + c

    @pl.loop(0, NCHUNK)
    def _fire(j):
      pltpu.async_copy(val_v, m_hbm.at[idx_v.at[j]], sem)

    @pl.loop(0, NCHUNK)
    def _drain(j):
      pltpu.make_async_copy(val_v, m_hbm.at[idx_v.at[0]], sem).wait()

  return scatter


_scatter_pos = _make_scatter(1.0)
_scatter_neg = _make_scatter(-1.0)

ROWS_BLK = 80


def _norm_body(m_blk, out_blk):
  x = m_blk[...].reshape(ROWS_BLK, NP)
  rs = jnp.sum(x, axis=1, keepdims=True)
  out_blk[...] = x[:, :N] / (rs + 1e-10)


_normalize = pl.pallas_call(
    _norm_body,
    out_shape=jax.ShapeDtypeStruct((N, N), jnp.float32),
    grid=(N // ROWS_BLK,),
    in_specs=[pl.BlockSpec((ROWS_BLK * NP,), lambda i: (i,))],
    out_specs=pl.BlockSpec((ROWS_BLK, N), lambda i: (i, 0)),
)


def _endpoints(edge_index):
  rows = jnp.concatenate([edge_index[0], edge_index[1]])
  cols = jnp.concatenate([edge_index[1], edge_index[0]])
  pad = PADDED - ENDP
  rows = jnp.concatenate([rows, jnp.broadcast_to(rows[-1:], (pad,))])
  cols = jnp.concatenate([cols, jnp.broadcast_to(cols[-1:], (pad,))])
  return rows.astype(jnp.int32), cols.astype(jnp.int32)


def kernel(pos_edge_index, neg_edge_index, num_nodes):
  rows_p, cols_p = _endpoints(pos_edge_index)
  rows_n, cols_n = _endpoints(neg_edge_index)
  m_ref = jax.new_ref(jnp.zeros((N * NP,), jnp.float32))
  _scatter_pos(rows_p, cols_p, m_ref)
  _scatter_neg(rows_n, cols_n, m_ref)
  m = jax.freeze(m_ref)
  return _normalize(m)
